# baseline (speedup 1.0000x reference)
import jax
import jax.numpy as jnp
from jax.experimental import pallas as pl
from jax.experimental.pallas import tpu as pltpu

B, N, D = 32, 576, 768
N_CLASSES, K = 200, 5
C = N_CLASSES + 1
CK = C * K
CKP = 1024
TEMPERATURE = 0.2

BG = 4
GB = B // BG
CT = 8
TCK = CKP // CT


def _proto_prep_kernel(p_ref, out_ref):
    p = p_ref[...]
    row = jax.lax.broadcasted_iota(jnp.int32, (CKP, 1), 0)
    valid = row < CK
    ssq = jnp.sum(p * p, axis=1, keepdims=True)
    inv = jax.lax.rsqrt(jnp.maximum(ssq, 1e-24))
    pn = jnp.where(valid, p * inv, 0.0)
    out_ref[...] = pn.astype(jnp.bfloat16)


def _token_prep_kernel(x_ref, out_ref):
    x = x_ref[...].astype(jnp.float32)
    ssq = jnp.sum(x * x, axis=1, keepdims=True)
    xn = (x * jax.lax.rsqrt(jnp.maximum(ssq, 1e-24))).astype(jnp.bfloat16)
    out_ref[...] = xn.T


def _main_kernel(pn_ref, xnT_ref, big_ref, imgp_ref):
    r = jax.lax.dot_general(
        pn_ref[...], xnT_ref[...], (((1,), (0,)), ((), ())),
        preferred_element_type=jnp.float32)
    r3 = r.reshape(TCK, GB, N)
    big_ref[...] = r3
    imgp_ref[...] = jnp.max(r3, axis=2).reshape(1, 1, TCK, GB)


def _epilogue_kernel(img_ref, saT_ref, imgT_ref, cls_ref):
    img = img_ref[...]
    saT = saT_ref[...]
    e = jnp.exp(saT - jnp.max(saT, axis=0, keepdims=True))
    s = e * (K / jnp.sum(e, axis=0, keepdims=True))
    j = jax.lax.broadcasted_iota(jnp.int32, (CKP, C), 0)
    c = jax.lax.broadcasted_iota(jnp.int32, (CKP, C), 1)
    cls = jnp.zeros((B, N_CLASSES), jnp.float32)
    planes = []
    for k in range(K):
        sel = (j == K * c + k).astype(jnp.float32)
        vk = jax.lax.dot_general(
            img, sel, (((1,), (0,)), ((), ())),
            preferred_element_type=jnp.float32)
        planes.append(vk)
        cls = cls + vk[:, :N_CLASSES] * s[k:k + 1, :]
    imgT_ref[...] = jnp.stack(planes, axis=0)
    cls_ref[...] = cls * (1.0 / TEMPERATURE)


def kernel(x, prototypes, sa_weights):
    x2d = x.reshape(B * N, D).astype(jnp.bfloat16)
    p2d = prototypes.reshape(CK, D)
    saT = sa_weights.T

    pn = pl.pallas_call(
        _proto_prep_kernel,
        grid=(1,),
        out_shape=jax.ShapeDtypeStruct((CKP, D), jnp.bfloat16),
        in_specs=[pl.BlockSpec((CKP, D), lambda i: (0, 0))],
        out_specs=pl.BlockSpec((CKP, D), lambda i: (0, 0)),
    )(p2d)

    xnT = pl.pallas_call(
        _token_prep_kernel,
        grid=(16,),
        out_shape=jax.ShapeDtypeStruct((D, B * N), jnp.bfloat16),
        in_specs=[pl.BlockSpec((1152, D), lambda i: (i, 0))],
        out_specs=pl.BlockSpec((D, 1152), lambda i: (0, i)),
    )(x2d)

    big, imgp = pl.pallas_call(
        _main_kernel,
        grid=(BG, CT),
        out_shape=(
            jax.ShapeDtypeStruct((CK, B, N), jnp.float32),
            jax.ShapeDtypeStruct((BG, CT, TCK, GB), jnp.float32),
        ),
        in_specs=[
            pl.BlockSpec((TCK, D), lambda g, t: (t, 0)),
            pl.BlockSpec((D, GB * N), lambda g, t: (0, g)),
        ],
        out_specs=(
            pl.BlockSpec((TCK, GB, N), lambda g, t: (t, g, 0)),
            pl.BlockSpec((1, 1, TCK, GB), lambda g, t: (g, t, 0, 0)),
        ),
        compiler_params=pltpu.CompilerParams(
            dimension_semantics=("arbitrary", "arbitrary")),
    )(pn, xnT)

    img2d = imgp.transpose(0, 3, 1, 2).reshape(B, CKP)

    imgT, cls = pl.pallas_call(
        _epilogue_kernel,
        grid=(1,),
        out_shape=(
            jax.ShapeDtypeStruct((K, B, C), jnp.float32),
            jax.ShapeDtypeStruct((B, N_CLASSES), jnp.float32),
        ),
        in_specs=[
            pl.BlockSpec((B, CKP), lambda i: (0, 0)),
            pl.BlockSpec((K, N_CLASSES), lambda i: (0, 0)),
        ],
        out_specs=(
            pl.BlockSpec((K, B, C), lambda i: (0, 0, 0)),
            pl.BlockSpec((B, N_CLASSES), lambda i: (0, 0)),
        ),
    )(img2d, saT)

    patch_prototype_logits = jnp.transpose(
        big.reshape(C, K, B, N), (2, 3, 0, 1))
    image_prototype_logits = jnp.transpose(imgT, (1, 2, 0))
    return (patch_prototype_logits, image_prototype_logits, cls)


# speedup vs baseline: 1.7378x; 1.7378x over previous
import jax
import jax.numpy as jnp
from jax.experimental import pallas as pl
from jax.experimental.pallas import tpu as pltpu

B, N, D = 32, 576, 768
N_CLASSES, K = 200, 5
C = N_CLASSES + 1
CK = C * K
CKP = 1024
TEMPERATURE = 0.2

BG = 4
GB = B // BG
CT = 4
TCK = CKP // CT
CHUNK = GB * N // CT
STEPS = BG * CT + CT


def _main_kernel(p_ref, x_ref, saT_ref, big_ref, imgT_ref, cls_ref,
                 xnT_scr, pn_scr, img_scr):
    s = pl.program_id(0)

    @pl.when(s == 0)
    def _proto_prep():
        p = p_ref[...]
        row = jax.lax.broadcasted_iota(jnp.int32, (CKP, 1), 0)
        ssq = jnp.sum(p * p, axis=1, keepdims=True)
        inv = jax.lax.rsqrt(jnp.maximum(ssq, 1e-24))
        pn_scr[...] = jnp.where(row < CK, p * inv, 0.0).astype(jnp.bfloat16)

    @pl.when(s < BG * CT)
    def _token_prep():
        pg2 = (s // CT) % 2
        xc = x_ref[...]
        ssq = jnp.sum(xc * xc, axis=1, keepdims=True)
        xn = (xc * jax.lax.rsqrt(jnp.maximum(ssq, 1e-24))
              ).astype(jnp.bfloat16)
        xnT_scr[pg2, :, pl.ds((s % CT) * CHUNK, CHUNK)] = xn.T

    @pl.when(s >= CT)
    def _mm():
        s4 = s - CT
        g = s4 // CT
        t = s4 % CT
        r = jax.lax.dot_general(
            pn_scr[pl.ds(t * TCK, TCK), :], xnT_scr[g % 2],
            (((1,), (0,)), ((), ())),
            preferred_element_type=jnp.float32)
        r3 = r.reshape(TCK, GB, N)
        big_ref[...] = r3
        img_scr[g, pl.ds(t * TCK, TCK), :] = jnp.max(r3, axis=2)

    @pl.when(s == STEPS - 1)
    def _epilogue():
        img = jnp.concatenate(
            [img_scr[g].T for g in range(BG)], axis=0)
        saT = saT_ref[...]
        e = jnp.exp(saT - jnp.max(saT, axis=0, keepdims=True))
        sw = e * (K / jnp.sum(e, axis=0, keepdims=True))
        j = jax.lax.broadcasted_iota(jnp.int32, (CKP, C), 0)
        c = jax.lax.broadcasted_iota(jnp.int32, (CKP, C), 1)
        cls = jnp.zeros((B, N_CLASSES), jnp.float32)
        planes = []
        for k in range(K):
            sel = (j == K * c + k).astype(jnp.float32)
            vk = jax.lax.dot_general(
                img, sel, (((1,), (0,)), ((), ())),
                preferred_element_type=jnp.float32)
            planes.append(vk)
            cls = cls + vk[:, :N_CLASSES] * sw[k:k + 1, :]
        imgT_ref[...] = jnp.stack(planes, axis=0)
        cls_ref[...] = cls * (1.0 / TEMPERATURE)


def kernel(x, prototypes, sa_weights):
    x2d = x.reshape(B * N, D)
    p2d = prototypes.reshape(CK, D)
    saT = sa_weights.T

    def _s4(s):
        return jnp.maximum(s - CT, 0)

    big, imgT, cls = pl.pallas_call(
        _main_kernel,
        grid=(STEPS,),
        out_shape=(
            jax.ShapeDtypeStruct((CK, B, N), jnp.float32),
            jax.ShapeDtypeStruct((K, B, C), jnp.float32),
            jax.ShapeDtypeStruct((B, N_CLASSES), jnp.float32),
        ),
        in_specs=[
            pl.BlockSpec((CKP, D), lambda s: (0, 0)),
            pl.BlockSpec((CHUNK, D),
                         lambda s: (jnp.minimum(s, BG * CT - 1), 0)),
            pl.BlockSpec((K, N_CLASSES), lambda s: (0, 0)),
        ],
        out_specs=(
            pl.BlockSpec((TCK, GB, N),
                         lambda s: (_s4(s) % CT, _s4(s) // CT, 0)),
            pl.BlockSpec((K, B, C), lambda s: (0, 0, 0)),
            pl.BlockSpec((B, N_CLASSES), lambda s: (0, 0)),
        ),
        scratch_shapes=[
            pltpu.VMEM((2, D, GB * N), jnp.bfloat16),
            pltpu.VMEM((CKP, D), jnp.bfloat16),
            pltpu.VMEM((BG, CKP, GB), jnp.float32),
        ],
        compiler_params=pltpu.CompilerParams(
            dimension_semantics=("arbitrary",)),
    )(p2d, x2d, saT)

    patch_prototype_logits = jnp.transpose(
        big.reshape(C, K, B, N), (2, 3, 0, 1))
    image_prototype_logits = jnp.transpose(imgT, (1, 2, 0))
    return (patch_prototype_logits, image_prototype_logits, cls)


# speedup vs baseline: 1.8500x; 1.0646x over previous
import jax
import jax.numpy as jnp
from jax.experimental import pallas as pl
from jax.experimental.pallas import tpu as pltpu

B, N, D = 32, 576, 768
N_CLASSES, K = 200, 5
C = N_CLASSES + 1
CK = C * K
CKP = 1024
TEMPERATURE = 0.2

BG = 4
GB = B // BG
CT = 4
TCK = CKP // CT
CHUNK = GB * N // CT
STEPS = BG * CT + CT


def _main_kernel(p_ref, x_ref, saT_ref, big_ref, imgT_ref, cls_ref,
                 xnT_scr, pn_scr, img_scr):
    s = pl.program_id(0)

    @pl.when(s == 0)
    def _proto_prep():
        p = p_ref[...]
        row = jax.lax.broadcasted_iota(jnp.int32, (CKP, 1), 0)
        ssq = jnp.sum(p * p, axis=1, keepdims=True)
        inv = jax.lax.rsqrt(jnp.maximum(ssq, 1e-24))
        pn_scr[...] = jnp.where(row < CK, p * inv, 0.0).astype(jnp.bfloat16)

    @pl.when(s < BG * CT)
    def _token_prep():
        pg2 = (s // CT) % 2
        xc = x_ref[...]
        ssq = jnp.sum(xc * xc, axis=1, keepdims=True)
        xn = (xc * jax.lax.rsqrt(jnp.maximum(ssq, 1e-24))
              ).astype(jnp.bfloat16)
        xnT_scr[pg2, pl.ds((s % CT) * CHUNK, CHUNK), :] = xn

    @pl.when(s >= CT)
    def _mm():
        s4 = s - CT
        g = s4 // CT
        t = s4 % CT
        r = jax.lax.dot_general(
            pn_scr[pl.ds(t * TCK, TCK), :], xnT_scr[g % 2],
            (((1,), (1,)), ((), ())),
            preferred_element_type=jnp.float32)
        r3 = r.reshape(TCK, GB, N)
        big_ref[...] = r3
        img_scr[g, pl.ds(t * TCK, TCK), :] = jnp.max(r3, axis=2)

    @pl.when(s == STEPS - 1)
    def _epilogue():
        img = jnp.concatenate(
            [img_scr[g].T for g in range(BG)], axis=0)
        saT = saT_ref[...]
        e = jnp.exp(saT - jnp.max(saT, axis=0, keepdims=True))
        sw = e * (K / jnp.sum(e, axis=0, keepdims=True))
        j = jax.lax.broadcasted_iota(jnp.int32, (CKP, C), 0)
        c = jax.lax.broadcasted_iota(jnp.int32, (CKP, C), 1)
        cls = jnp.zeros((B, N_CLASSES), jnp.float32)
        planes = []
        for k in range(K):
            sel = (j == K * c + k).astype(jnp.float32)
            vk = jax.lax.dot_general(
                img, sel, (((1,), (0,)), ((), ())),
                preferred_element_type=jnp.float32)
            planes.append(vk)
            cls = cls + vk[:, :N_CLASSES] * sw[k:k + 1, :]
        imgT_ref[...] = jnp.stack(planes, axis=0)
        cls_ref[...] = cls * (1.0 / TEMPERATURE)


def kernel(x, prototypes, sa_weights):
    x2d = x.reshape(B * N, D)
    p2d = prototypes.reshape(CK, D)
    saT = sa_weights.T

    def _s4(s):
        return jnp.maximum(s - CT, 0)

    big, imgT, cls = pl.pallas_call(
        _main_kernel,
        grid=(STEPS,),
        out_shape=(
            jax.ShapeDtypeStruct((CK, B, N), jnp.float32),
            jax.ShapeDtypeStruct((K, B, C), jnp.float32),
            jax.ShapeDtypeStruct((B, N_CLASSES), jnp.float32),
        ),
        in_specs=[
            pl.BlockSpec((CKP, D), lambda s: (0, 0)),
            pl.BlockSpec((CHUNK, D),
                         lambda s: (jnp.minimum(s, BG * CT - 1), 0)),
            pl.BlockSpec((K, N_CLASSES), lambda s: (0, 0)),
        ],
        out_specs=(
            pl.BlockSpec((TCK, GB, N),
                         lambda s: (_s4(s) % CT, _s4(s) // CT, 0)),
            pl.BlockSpec((K, B, C), lambda s: (0, 0, 0)),
            pl.BlockSpec((B, N_CLASSES), lambda s: (0, 0)),
        ),
        scratch_shapes=[
            pltpu.VMEM((2, GB * N, D), jnp.bfloat16),
            pltpu.VMEM((CKP, D), jnp.bfloat16),
            pltpu.VMEM((BG, CKP, GB), jnp.float32),
        ],
        compiler_params=pltpu.CompilerParams(
            dimension_semantics=("arbitrary",)),
    )(p2d, x2d, saT)

    patch_prototype_logits = jnp.transpose(
        big.reshape(C, K, B, N), (2, 3, 0, 1))
    image_prototype_logits = jnp.transpose(imgT, (1, 2, 0))
    return (patch_prototype_logits, image_prototype_logits, cls)
